# baseline (speedup 1.0000x reference)
import functools

import jax
import jax.numpy as jnp
from jax import lax
from jax.experimental import pallas as pl
from jax.experimental.pallas import tpu as pltpu
from jax.experimental.pallas import tpu_sc as plsc

N = 10000
WIDTH = 32
KER_IN = 4
DEPTH = 2

NC = 2
NS = 16
NW = NC * NS
N_PAD = 10240
RPS = N_PAD // NS
HW = 128
TE = 512
ROWT = 1000

_pallas_call = pl.pallas_call


def _mesh():
    return plsc.VectorSubcoreMesh(core_axis_name="c", subcore_axis_name="s",
                                  num_cores=NC, num_subcores=NS)


def _sc_gather2(h, idx1, idx2):
    e1, e2 = idx1.shape[0], idx2.shape[0]
    rows1 = e1 // NW // 128
    rows2 = e2 // NW // 128
    chw = max(rows1, rows2) * 128

    @functools.partial(
        pl.kernel, mesh=_mesh(),
        out_type=[jax.ShapeDtypeStruct((e1, HW), jnp.float32),
                  jax.ShapeDtypeStruct((e2, HW), jnp.float32)],
        scratch_types=[pltpu.VMEM((chw,), jnp.int32)]
                      + [pltpu.VMEM((128, HW), jnp.float32)] * 4
                      + [pltpu.SemaphoreType.DMA] * 8)
    def k(h_hbm, idx1_hbm, idx2_hbm, o1_hbm, o2_hbm, idx_v, b0, b1, b2, b3,
          g0, g1, g2, g3, w0, w1, w2, w3):
        wid = lax.axis_index("s") * NC + lax.axis_index("c")
        bufs = (b0, b1, b2, b3)
        gs = (g0, g1, g2, g3)
        ws = (w0, w1, w2, w3)

        def phase(idx_hbm, out_hbm, rows):
            base = wid * rows
            pltpu.sync_copy(idx_hbm.at[pl.ds(base * 128, rows * 128)],
                            idx_v.at[pl.ds(0, rows * 128)])

            def g_desc(j, s):
                return pltpu.make_async_copy(
                    h_hbm.at[idx_v.at[pl.ds(j * 128, 128)]], bufs[s], gs[s])

            def w_desc(j, s):
                return pltpu.make_async_copy(
                    bufs[s], out_hbm.at[pl.ds((base + j) * 128, 128)], ws[s])

            for s in range(4):
                g_desc(s, s).start()

            q_n = rows // 4

            @pl.loop(0, q_n)
            def _(q):
                for s in range(4):
                    g_desc(q * 4 + s, s).wait()
                    w_desc(q * 4 + s, s).start()
                for s in range(4):
                    w_desc(q * 4 + s, s).wait()

                    @pl.when(q < q_n - 1)
                    def _():
                        g_desc(q * 4 + 4 + s, s).start()

        phase(idx1_hbm, o1_hbm, rows1)
        phase(idx2_hbm, o2_hbm, rows2)

    return k(h, idx1, idx2)


def _sc_scatter_add2(m1, m2, idx1, idx2):
    rows1 = idx1.shape[0] // NW // 128
    rows2 = idx2.shape[0] // NW // 128
    chw = max(rows1, rows2) * 128
    zeros = jnp.zeros((RPS, HW), jnp.float32)
    out_t = jax.ShapeDtypeStruct((NC, N_PAD, HW), jnp.float32)

    @functools.partial(
        pl.kernel, mesh=_mesh(),
        out_type=[out_t, out_t],
        scratch_types=[pltpu.VMEM_SHARED((N_PAD, HW), jnp.float32),
                       pltpu.VMEM((chw,), jnp.int32)]
                      + [pltpu.VMEM((128, HW), jnp.float32)] * 2
                      + [pltpu.SemaphoreType.DMA] * 4)
    def k(m1_hbm, m2_hbm, idx1_hbm, idx2_hbm, z_hbm, o1_hbm, o2_hbm,
          acc, idx_v, b0, b1, f0, f1, a0, a1):
        cc = lax.axis_index("c")
        ss = lax.axis_index("s")
        wid = ss * NC + cc
        sl = pl.ds(ss * RPS, RPS)
        bufs = (b0, b1)
        fs = (f0, f1)
        as_ = (a0, a1)

        def phase(msg_hbm, idx_hbm, out_hbm, rows):
            base = wid * rows
            pltpu.sync_copy(z_hbm, acc.at[sl])
            pltpu.sync_copy(idx_hbm.at[pl.ds(base * 128, rows * 128)],
                            idx_v.at[pl.ds(0, rows * 128)])
            plsc.subcore_barrier()

            def f_desc(j, s):
                return pltpu.make_async_copy(
                    msg_hbm.at[pl.ds((base + j) * 128, 128)], bufs[s], fs[s])

            def a_desc(j, s):
                return pltpu.make_async_copy(
                    bufs[s], acc.at[idx_v.at[pl.ds(j * 128, 128)]], as_[s])

            for s in range(2):
                f_desc(s, s).start()

            q_n = rows // 2

            @pl.loop(0, q_n)
            def _(q):
                for s in range(2):
                    f_desc(q * 2 + s, s).wait()
                    a_desc(q * 2 + s, s).start(add=True)
                for s in range(2):
                    a_desc(q * 2 + s, s).wait()

                    @pl.when(q < q_n - 1)
                    def _():
                        f_desc(q * 2 + 2 + s, s).start()

            plsc.subcore_barrier()
            pltpu.sync_copy(acc.at[sl], out_hbm.at[cc].at[sl])
            plsc.subcore_barrier()

        phase(m1_hbm, idx1_hbm, o1_hbm, rows1)
        phase(m2_hbm, idx2_hbm, o2_hbm, rows2)

    return k(m1, m2, idx1, idx2, zeros)


def _msg_body(ea_ref, xg_ref, w1_ref, b1_ref, w2_ref, b2_ref, w3_ref, b3_ref,
              r_ref, s_ref, out_ref):
    f32 = jnp.float32
    h1 = jnp.maximum(jnp.dot(ea_ref[...], w1_ref[...],
                             preferred_element_type=f32) + b1_ref[...], 0.0)
    h2 = jnp.maximum(jnp.dot(h1, w2_ref[...],
                             preferred_element_type=f32) + b2_ref[...], 0.0)
    w = jnp.dot(h2.astype(jnp.bfloat16), w3_ref[...].astype(jnp.bfloat16),
                preferred_element_type=f32) + b3_ref[...]
    xb = jnp.dot(xg_ref[...], r_ref[...], preferred_element_type=f32)
    pv = w * xb
    pv = pv[:, :512] + pv[:, 512:]
    pv = pv[:, :256] + pv[:, 256:]
    pv = pv[:, :128] + pv[:, 128:]
    m = jnp.dot(pv, s_ref[...], preferred_element_type=f32)
    te = m.shape[0]
    out_ref[...] = jnp.concatenate(
        [m, jnp.ones((te, 1), f32), jnp.zeros((te, HW - WIDTH - 1), f32)],
        axis=1)


def _msg_call(eap, xg, ker, rm, sm):
    (w1, b1), (w2, b2), (w3, b3) = ker
    e_pad = eap.shape[0]
    full = lambda a: pl.BlockSpec(a.shape, lambda i: tuple(0 for _ in a.shape))
    args = (eap, xg, w1, b1[None], w2, b2[None], w3, b3[None], rm, sm)
    return _pallas_call(
        _msg_body,
        grid=(e_pad // TE,),
        in_specs=[pl.BlockSpec((TE, KER_IN), lambda i: (i, 0)),
                  pl.BlockSpec((TE, HW), lambda i: (i, 0))]
                 + [full(a) for a in args[2:]],
        out_specs=pl.BlockSpec((TE, HW), lambda i: (i, 0)),
        out_shape=jax.ShapeDtypeStruct((e_pad, HW), jnp.float32),
    )(*args)


def _wide(res):
    return jnp.concatenate(
        [res, jnp.zeros((res.shape[0], HW - WIDTH), res.dtype)], axis=1)


def _x0_body(x_ref, w_ref, b_ref, out_ref):
    out_ref[...] = _wide(x_ref[...] * w_ref[...] + b_ref[...])


def _x0_call(x, w, b):
    return _pallas_call(
        _x0_body,
        grid=(N // ROWT,),
        in_specs=[pl.BlockSpec((ROWT, 1), lambda i: (i, 0)),
                  pl.BlockSpec((1, WIDTH), lambda i: (0, 0)),
                  pl.BlockSpec((1, WIDTH), lambda i: (0, 0))],
        out_specs=pl.BlockSpec((ROWT, HW), lambda i: (i, 0)),
        out_shape=jax.ShapeDtypeStruct((N, HW), jnp.float32),
    )(x, w[None, 0], b[None])


def _comb_body(h_ref, x0_ref, s1a, s1b, s2a, s2b,
               r1_ref, z1_ref, r2_ref, z2_ref, out_ref):
    f32 = jnp.float32
    h = h_ref[:, :WIDTH]
    t1 = s1a[...] + s1b[...]
    t2 = s2a[...] + s2b[...]
    agg1 = t1[:, :WIDTH] / jnp.maximum(t1[:, WIDTH:WIDTH + 1], 1.0)
    agg2 = t2[:, :WIDTH] / jnp.maximum(t2[:, WIDTH:WIDTH + 1], 1.0)
    t = (agg1 + jnp.dot(h, r1_ref[...], preferred_element_type=f32) + z1_ref[...]
         + agg2 + jnp.dot(h, r2_ref[...], preferred_element_type=f32) + z2_ref[...])
    out_ref[...] = _wide(jnp.maximum(t, 0.0) + x0_ref[:, :WIDTH])


def _comb_call(h, x0, s1, s2, r1, z1, r2, z2):
    full = lambda a: pl.BlockSpec(a.shape, lambda i: tuple(0 for _ in a.shape))
    rowh = pl.BlockSpec((ROWT, HW), lambda i: (i, 0))
    args = (h, x0, s1[0], s1[1], s2[0], s2[1], r1, z1[None], r2, z2[None])
    return _pallas_call(
        _comb_body,
        grid=(N // ROWT,),
        in_specs=[rowh, rowh, rowh, rowh, rowh, rowh]
                 + [full(a) for a in args[6:]],
        out_specs=rowh,
        out_shape=jax.ShapeDtypeStruct((N, HW), jnp.float32),
    )(*args)


def _epi_body(h_ref, w2_ref, b2_ref, w3_ref, b3_ref, out_ref):
    f32 = jnp.float32
    t = jnp.maximum(jnp.dot(h_ref[:, :WIDTH], w2_ref[...],
                            preferred_element_type=f32) + b2_ref[...], 0.0)
    out_ref[...] = jnp.dot(t, w3_ref[...], preferred_element_type=f32) + b3_ref[...]


def _epi_call(h, w2, b2, w3, b3):
    full = lambda a: pl.BlockSpec(a.shape, lambda i: tuple(0 for _ in a.shape))
    args = (h, w2, b2[None], w3, b3[None])
    return _pallas_call(
        _epi_body,
        grid=(N // ROWT,),
        in_specs=[pl.BlockSpec((ROWT, HW), lambda i: (i, 0))]
                 + [full(a) for a in args[1:]],
        out_specs=pl.BlockSpec((ROWT, 1), lambda i: (i, 0)),
        out_shape=jax.ShapeDtypeStruct((N, 1), jnp.float32),
    )(*args)


def _prep_edges(edge_index):
    e0 = edge_index.shape[1]
    e_pad = -(-e0 // (NW * 128)) * (NW * 128)
    src = jnp.concatenate(
        [edge_index[0], jnp.zeros((e_pad - e0,), jnp.int32)])
    dst = jnp.concatenate(
        [edge_index[1], jnp.full((e_pad - e0,), N, jnp.int32)])
    return e_pad, src, dst


def kernel(x, edge_attr, edge_attr_boundary, params, edge_index,
           edge_index_boundary):
    p = params
    f32 = jnp.float32

    e_pad1, src1, dst1 = _prep_edges(edge_index)
    e_pad2, src2, dst2 = _prep_edges(edge_index_boundary)
    ea1 = jnp.concatenate(
        [edge_attr, jnp.zeros((e_pad1 - edge_attr.shape[0], KER_IN), f32)])
    ea2 = jnp.concatenate(
        [edge_attr_boundary,
         jnp.zeros((e_pad2 - edge_attr_boundary.shape[0], KER_IN), f32)])

    rm = (jnp.arange(32 * WIDTH)[None, :] // WIDTH
          == jnp.arange(HW)[:, None]).astype(f32)
    sm = (jnp.arange(128)[:, None] % WIDTH
          == jnp.arange(WIDTH)[None, :]).astype(f32)

    x0 = _x0_call(x, p["fc1_w"], p["fc1_b"])
    h = x0
    for _ in range(DEPTH):
        xg1, xg2 = _sc_gather2(h, src1, src2)
        m1 = _msg_call(ea1, xg1, p["ker1"], rm, sm)
        m2 = _msg_call(ea2, xg2, p["ker2"], rm, sm)
        s1, s2 = _sc_scatter_add2(m1, m2, dst1, dst2)
        h = _comb_call(h, x0, s1, s2,
                       p["root1"], p["bias1"], p["root2"], p["bias2"])

    return _epi_call(h, p["fc2_w"], p["fc2_b"], p["fc3_w"], p["fc3_b"])


# speedup vs baseline: 1.1077x; 1.1077x over previous
import functools

import jax
import jax.numpy as jnp
from jax import lax
from jax.experimental import pallas as pl
from jax.experimental.pallas import tpu as pltpu
from jax.experimental.pallas import tpu_sc as plsc

N = 10000
WIDTH = 32
KER_IN = 4
DEPTH = 2

NC = 2
NS = 16
NW = NC * NS
N_PAD = 10240
RPS = N_PAD // NS
HW = 128
TE = 512
ROWT = 1000

_pallas_call = pl.pallas_call


def _mesh():
    return plsc.VectorSubcoreMesh(core_axis_name="c", subcore_axis_name="s",
                                  num_cores=NC, num_subcores=NS)


def _sc_gather(h, idx):
    e_pad = idx.shape[0]
    rows = e_pad // NW // 128
    chw = rows * 128
    q_n = rows // 4

    @functools.partial(
        pl.kernel, mesh=_mesh(),
        out_type=jax.ShapeDtypeStruct((e_pad, HW), jnp.float32),
        scratch_types=[pltpu.VMEM((chw,), jnp.int32)]
                      + [pltpu.VMEM((128, HW), jnp.float32)] * 4
                      + [pltpu.SemaphoreType.DMA] * 8)
    def k(h_hbm, idx_hbm, out_hbm, idx_v, b0, b1, b2, b3,
          g0, g1, g2, g3, w0, w1, w2, w3):
        wid = lax.axis_index("s") * NC + lax.axis_index("c")
        base = wid * rows
        bufs = (b0, b1, b2, b3)
        gs = (g0, g1, g2, g3)
        ws = (w0, w1, w2, w3)
        pltpu.sync_copy(idx_hbm.at[pl.ds(base * 128, chw)], idx_v)

        def g_desc(j, s):
            return pltpu.make_async_copy(
                h_hbm.at[idx_v.at[pl.ds(j * 128, 128)]], bufs[s], gs[s])

        def w_desc(j, s):
            return pltpu.make_async_copy(
                bufs[s], out_hbm.at[pl.ds((base + j) * 128, 128)], ws[s])

        for s in range(4):
            g_desc(s, s).start()

        @pl.loop(0, q_n)
        def _(q):
            for s in range(4):
                g_desc(q * 4 + s, s).wait()
                w_desc(q * 4 + s, s).start()
            for s in range(4):
                w_desc(q * 4 + s, s).wait()

                @pl.when(q < q_n - 1)
                def _():
                    g_desc(q * 4 + 4 + s, s).start()

    return k(h, idx)


def _sc_scatter_add(msg, idx):
    e_pad = idx.shape[0]
    rows = e_pad // NW // 128
    chw = rows * 128
    q_n = rows // 2
    zeros = jnp.zeros((RPS, HW), jnp.float32)

    @functools.partial(
        pl.kernel, mesh=_mesh(),
        out_type=jax.ShapeDtypeStruct((NC, N_PAD, HW), jnp.float32),
        scratch_types=[pltpu.VMEM_SHARED((N_PAD, HW), jnp.float32),
                       pltpu.VMEM((chw,), jnp.int32)]
                      + [pltpu.VMEM((128, HW), jnp.float32)] * 2
                      + [pltpu.SemaphoreType.DMA] * 4)
    def k(msg_hbm, idx_hbm, z_hbm, out_hbm, acc, idx_v, b0, b1,
          f0, f1, a0, a1):
        cc = lax.axis_index("c")
        ss = lax.axis_index("s")
        wid = ss * NC + cc
        sl = pl.ds(ss * RPS, RPS)
        base = wid * rows
        bufs = (b0, b1)
        fs = (f0, f1)
        as_ = (a0, a1)
        pltpu.sync_copy(z_hbm, acc.at[sl])
        pltpu.sync_copy(idx_hbm.at[pl.ds(base * 128, chw)], idx_v)
        plsc.subcore_barrier()

        def f_desc(j, s):
            return pltpu.make_async_copy(
                msg_hbm.at[pl.ds((base + j) * 128, 128)], bufs[s], fs[s])

        def a_desc(j, s):
            return pltpu.make_async_copy(
                bufs[s], acc.at[idx_v.at[pl.ds(j * 128, 128)]], as_[s])

        for s in range(2):
            f_desc(s, s).start()

        @pl.loop(0, q_n)
        def _(q):
            for s in range(2):
                f_desc(q * 2 + s, s).wait()
                a_desc(q * 2 + s, s).start(add=True)
            for s in range(2):
                a_desc(q * 2 + s, s).wait()

                @pl.when(q < q_n - 1)
                def _():
                    f_desc(q * 2 + 2 + s, s).start()

        plsc.subcore_barrier()
        pltpu.sync_copy(acc.at[sl], out_hbm.at[cc].at[sl])

    return k(msg, idx, zeros)


def _msg_body(ea_ref, xg_ref, w1_ref, b1_ref, w2_ref, b2_ref, w3_ref, b3_ref,
              r_ref, s_ref, out_ref):
    f32 = jnp.float32
    h1 = jnp.maximum(jnp.dot(ea_ref[...], w1_ref[...],
                             preferred_element_type=f32) + b1_ref[...], 0.0)
    h2 = jnp.maximum(jnp.dot(h1, w2_ref[...],
                             preferred_element_type=f32) + b2_ref[...], 0.0)
    w = jnp.dot(h2.astype(jnp.bfloat16), w3_ref[...].astype(jnp.bfloat16),
                preferred_element_type=f32) + b3_ref[...]
    xb = jnp.dot(xg_ref[...], r_ref[...], preferred_element_type=f32)
    pv = w * xb
    pv = pv[:, :512] + pv[:, 512:]
    pv = pv[:, :256] + pv[:, 256:]
    pv = pv[:, :128] + pv[:, 128:]
    m = jnp.dot(pv, s_ref[...], preferred_element_type=f32)
    te = m.shape[0]
    out_ref[...] = jnp.concatenate(
        [m, jnp.ones((te, 1), f32), jnp.zeros((te, HW - WIDTH - 1), f32)],
        axis=1)


def _msg_call(eap, xg, ker, rm, sm):
    (w1, b1), (w2, b2), (w3, b3) = ker
    e_pad = eap.shape[0]
    full = lambda a: pl.BlockSpec(a.shape, lambda i: tuple(0 for _ in a.shape))
    args = (eap, xg, w1, b1[None], w2, b2[None], w3, b3[None], rm, sm)
    return _pallas_call(
        _msg_body,
        grid=(e_pad // TE,),
        in_specs=[pl.BlockSpec((TE, KER_IN), lambda i: (i, 0)),
                  pl.BlockSpec((TE, HW), lambda i: (i, 0))]
                 + [full(a) for a in args[2:]],
        out_specs=pl.BlockSpec((TE, HW), lambda i: (i, 0)),
        out_shape=jax.ShapeDtypeStruct((e_pad, HW), jnp.float32),
    )(*args)


def _wide(res):
    return jnp.concatenate(
        [res, jnp.zeros((res.shape[0], HW - WIDTH), res.dtype)], axis=1)


def _x0_body(x_ref, w_ref, b_ref, out_ref):
    out_ref[...] = _wide(x_ref[...] * w_ref[...] + b_ref[...])


def _x0_call(x, w, b):
    return _pallas_call(
        _x0_body,
        grid=(N // ROWT,),
        in_specs=[pl.BlockSpec((ROWT, 1), lambda i: (i, 0)),
                  pl.BlockSpec((1, WIDTH), lambda i: (0, 0)),
                  pl.BlockSpec((1, WIDTH), lambda i: (0, 0))],
        out_specs=pl.BlockSpec((ROWT, HW), lambda i: (i, 0)),
        out_shape=jax.ShapeDtypeStruct((N, HW), jnp.float32),
    )(x, w[None, 0], b[None])


def _comb_body(h_ref, x0_ref, s1a, s1b, s2a, s2b,
               r1_ref, z1_ref, r2_ref, z2_ref, out_ref):
    f32 = jnp.float32
    h = h_ref[:, :WIDTH]
    t1 = s1a[...] + s1b[...]
    t2 = s2a[...] + s2b[...]
    agg1 = t1[:, :WIDTH] / jnp.maximum(t1[:, WIDTH:WIDTH + 1], 1.0)
    agg2 = t2[:, :WIDTH] / jnp.maximum(t2[:, WIDTH:WIDTH + 1], 1.0)
    t = (agg1 + jnp.dot(h, r1_ref[...], preferred_element_type=f32) + z1_ref[...]
         + agg2 + jnp.dot(h, r2_ref[...], preferred_element_type=f32) + z2_ref[...])
    out_ref[...] = _wide(jnp.maximum(t, 0.0) + x0_ref[:, :WIDTH])


def _comb_call(h, x0, s1, s2, r1, z1, r2, z2):
    full = lambda a: pl.BlockSpec(a.shape, lambda i: tuple(0 for _ in a.shape))
    rowh = pl.BlockSpec((ROWT, HW), lambda i: (i, 0))
    args = (h, x0, s1[0], s1[1], s2[0], s2[1], r1, z1[None], r2, z2[None])
    return _pallas_call(
        _comb_body,
        grid=(N // ROWT,),
        in_specs=[rowh, rowh, rowh, rowh, rowh, rowh]
                 + [full(a) for a in args[6:]],
        out_specs=rowh,
        out_shape=jax.ShapeDtypeStruct((N, HW), jnp.float32),
    )(*args)


def _epi_body(h_ref, w2_ref, b2_ref, w3_ref, b3_ref, out_ref):
    f32 = jnp.float32
    t = jnp.maximum(jnp.dot(h_ref[:, :WIDTH], w2_ref[...],
                            preferred_element_type=f32) + b2_ref[...], 0.0)
    out_ref[...] = jnp.dot(t, w3_ref[...], preferred_element_type=f32) + b3_ref[...]


def _epi_call(h, w2, b2, w3, b3):
    full = lambda a: pl.BlockSpec(a.shape, lambda i: tuple(0 for _ in a.shape))
    args = (h, w2, b2[None], w3, b3[None])
    return _pallas_call(
        _epi_body,
        grid=(N // ROWT,),
        in_specs=[pl.BlockSpec((ROWT, HW), lambda i: (i, 0))]
                 + [full(a) for a in args[1:]],
        out_specs=pl.BlockSpec((ROWT, 1), lambda i: (i, 0)),
        out_shape=jax.ShapeDtypeStruct((N, 1), jnp.float32),
    )(*args)


def _prep_edges(edge_index):
    e0 = edge_index.shape[1]
    e_pad = -(-e0 // (NW * 128)) * (NW * 128)
    src = jnp.concatenate(
        [edge_index[0], jnp.zeros((e_pad - e0,), jnp.int32)])
    dst = jnp.concatenate(
        [edge_index[1], jnp.full((e_pad - e0,), N, jnp.int32)])
    return e_pad, src, dst


def kernel(x, edge_attr, edge_attr_boundary, params, edge_index,
           edge_index_boundary):
    p = params
    f32 = jnp.float32

    e_pad1, src1, dst1 = _prep_edges(edge_index)
    e_pad2, src2, dst2 = _prep_edges(edge_index_boundary)
    ea1 = jnp.concatenate(
        [edge_attr, jnp.zeros((e_pad1 - edge_attr.shape[0], KER_IN), f32)])
    ea2 = jnp.concatenate(
        [edge_attr_boundary,
         jnp.zeros((e_pad2 - edge_attr_boundary.shape[0], KER_IN), f32)])

    rm = (jnp.arange(32 * WIDTH)[None, :] // WIDTH
          == jnp.arange(HW)[:, None]).astype(f32)
    sm = (jnp.arange(128)[:, None] % WIDTH
          == jnp.arange(WIDTH)[None, :]).astype(f32)

    x0 = _x0_call(x, p["fc1_w"], p["fc1_b"])
    h = x0
    for _ in range(DEPTH):
        xg1 = _sc_gather(h, src1)
        xg2 = _sc_gather(h, src2)
        m1 = _msg_call(ea1, xg1, p["ker1"], rm, sm)
        m2 = _msg_call(ea2, xg2, p["ker2"], rm, sm)
        s1 = _sc_scatter_add(m1, dst1)
        s2 = _sc_scatter_add(m2, dst2)
        h = _comb_call(h, x0, s1, s2,
                       p["root1"], p["bias1"], p["root2"], p["bias2"])

    return _epi_call(h, p["fc2_w"], p["fc2_b"], p["fc3_w"], p["fc3_b"])


# speedup vs baseline: 1.1098x; 1.0019x over previous
import functools

import jax
import jax.numpy as jnp
from jax import lax
from jax.experimental import pallas as pl
from jax.experimental.pallas import tpu as pltpu
from jax.experimental.pallas import tpu_sc as plsc

N = 10000
WIDTH = 32
KER_IN = 4
DEPTH = 2

NC = 2
NS = 16
NW = NC * NS
N_PAD = 10240
RPS = N_PAD // NS
HW = 128
TE = 512
ROWT = 1000

_pallas_call = pl.pallas_call


def _mesh():
    return plsc.VectorSubcoreMesh(core_axis_name="c", subcore_axis_name="s",
                                  num_cores=NC, num_subcores=NS)


def _sc_gather(h, idx):
    e_pad = idx.shape[0]
    rows = e_pad // NW // 128
    chw = rows * 128
    q_n = rows // 4

    @functools.partial(
        pl.kernel, mesh=_mesh(),
        out_type=jax.ShapeDtypeStruct((e_pad, HW), jnp.float32),
        scratch_types=[pltpu.VMEM((chw,), jnp.int32)]
                      + [pltpu.VMEM((128, HW), jnp.float32)] * 4
                      + [pltpu.SemaphoreType.DMA] * 8)
    def k(h_hbm, idx_hbm, out_hbm, idx_v, b0, b1, b2, b3,
          g0, g1, g2, g3, w0, w1, w2, w3):
        wid = lax.axis_index("s") * NC + lax.axis_index("c")
        base = wid * rows
        bufs = (b0, b1, b2, b3)
        gs = (g0, g1, g2, g3)
        ws = (w0, w1, w2, w3)
        pltpu.sync_copy(idx_hbm.at[pl.ds(base * 128, chw)], idx_v)

        def g_desc(j, s):
            return pltpu.make_async_copy(
                h_hbm.at[idx_v.at[pl.ds(j * 128, 128)]], bufs[s], gs[s])

        def w_desc(j, s):
            return pltpu.make_async_copy(
                bufs[s], out_hbm.at[pl.ds((base + j) * 128, 128)], ws[s])

        for s in range(4):
            g_desc(s, s).start()

        @pl.loop(0, q_n)
        def _(q):
            for s in range(4):
                g_desc(q * 4 + s, s).wait()
                w_desc(q * 4 + s, s).start()
            for s in range(4):
                w_desc(q * 4 + s, s).wait()

                @pl.when(q < q_n - 1)
                def _():
                    g_desc(q * 4 + 4 + s, s).start()

    return k(h, idx)


def _sc_scatter_add(msg, idx):
    e_pad = idx.shape[0]
    rows = e_pad // NW // 128
    chw = rows * 128
    q_n = rows // 2
    zeros = jnp.zeros((RPS, HW), jnp.float32)

    @functools.partial(
        pl.kernel, mesh=_mesh(),
        out_type=jax.ShapeDtypeStruct((NC, N_PAD, HW), jnp.float32),
        scratch_types=[pltpu.VMEM_SHARED((N_PAD, HW), jnp.float32),
                       pltpu.VMEM((chw,), jnp.int32)]
                      + [pltpu.VMEM((128, HW), jnp.float32)] * 2
                      + [pltpu.SemaphoreType.DMA] * 4)
    def k(msg_hbm, idx_hbm, z_hbm, out_hbm, acc, idx_v, b0, b1,
          f0, f1, a0, a1):
        cc = lax.axis_index("c")
        ss = lax.axis_index("s")
        wid = ss * NC + cc
        sl = pl.ds(ss * RPS, RPS)
        base = wid * rows
        bufs = (b0, b1)
        fs = (f0, f1)
        as_ = (a0, a1)
        pltpu.sync_copy(z_hbm, acc.at[sl])
        pltpu.sync_copy(idx_hbm.at[pl.ds(base * 128, chw)], idx_v)
        plsc.subcore_barrier()

        def f_desc(j, s):
            return pltpu.make_async_copy(
                msg_hbm.at[pl.ds((base + j) * 128, 128)], bufs[s], fs[s])

        def a_desc(j, s):
            return pltpu.make_async_copy(
                bufs[s], acc.at[idx_v.at[pl.ds(j * 128, 128)]], as_[s])

        for s in range(2):
            f_desc(s, s).start()

        @pl.loop(0, q_n)
        def _(q):
            for s in range(2):
                f_desc(q * 2 + s, s).wait()
                a_desc(q * 2 + s, s).start(add=True)
            for s in range(2):
                a_desc(q * 2 + s, s).wait()

                @pl.when(q < q_n - 1)
                def _():
                    f_desc(q * 2 + 2 + s, s).start()

        plsc.subcore_barrier()
        pltpu.sync_copy(acc.at[sl], out_hbm.at[cc].at[sl])

    return k(msg, idx, zeros)


def _msg_body(ea_ref, xg_ref, w1_ref, b1_ref, w2_ref, b2_ref, w3_ref, b3_ref,
              r_ref, s_ref, out_ref):
    f32 = jnp.float32
    h1 = jnp.maximum(jnp.dot(ea_ref[...], w1_ref[...],
                             preferred_element_type=f32) + b1_ref[...], 0.0)
    h2 = jnp.maximum(jnp.dot(h1, w2_ref[...],
                             preferred_element_type=f32) + b2_ref[...], 0.0)
    w = jnp.dot(h2, w3_ref[...], preferred_element_type=f32) + b3_ref[...]
    xb = jnp.dot(xg_ref[...], r_ref[...], preferred_element_type=f32)
    pv = w * xb
    pv = pv[:, :512] + pv[:, 512:]
    pv = pv[:, :256] + pv[:, 256:]
    pv = pv[:, :128] + pv[:, 128:]
    m = jnp.dot(pv, s_ref[...], preferred_element_type=f32)
    te = m.shape[0]
    out_ref[...] = jnp.concatenate(
        [m, jnp.ones((te, 1), f32), jnp.zeros((te, HW - WIDTH - 1), f32)],
        axis=1)


def _msg_call(eap, xg, ker, rm, sm):
    (w1, b1), (w2, b2), (w3, b3) = ker
    e_pad = eap.shape[0]
    full = lambda a: pl.BlockSpec(a.shape, lambda i: tuple(0 for _ in a.shape))
    args = (eap, xg, w1, b1[None], w2, b2[None], w3, b3[None], rm, sm)
    return _pallas_call(
        _msg_body,
        grid=(e_pad // TE,),
        in_specs=[pl.BlockSpec((TE, KER_IN), lambda i: (i, 0)),
                  pl.BlockSpec((TE, HW), lambda i: (i, 0))]
                 + [full(a) for a in args[2:]],
        out_specs=pl.BlockSpec((TE, HW), lambda i: (i, 0)),
        out_shape=jax.ShapeDtypeStruct((e_pad, HW), jnp.float32),
    )(*args)


def _wide(res):
    return jnp.concatenate(
        [res, jnp.zeros((res.shape[0], HW - WIDTH), res.dtype)], axis=1)


def _x0_body(x_ref, w_ref, b_ref, out_ref):
    out_ref[...] = _wide(x_ref[...] * w_ref[...] + b_ref[...])


def _x0_call(x, w, b):
    return _pallas_call(
        _x0_body,
        grid=(N // ROWT,),
        in_specs=[pl.BlockSpec((ROWT, 1), lambda i: (i, 0)),
                  pl.BlockSpec((1, WIDTH), lambda i: (0, 0)),
                  pl.BlockSpec((1, WIDTH), lambda i: (0, 0))],
        out_specs=pl.BlockSpec((ROWT, HW), lambda i: (i, 0)),
        out_shape=jax.ShapeDtypeStruct((N, HW), jnp.float32),
    )(x, w[None, 0], b[None])


def _comb_body(h_ref, x0_ref, s1a, s1b, s2a, s2b,
               r1_ref, z1_ref, r2_ref, z2_ref, out_ref):
    f32 = jnp.float32
    h = h_ref[:, :WIDTH]
    t1 = s1a[...] + s1b[...]
    t2 = s2a[...] + s2b[...]
    agg1 = t1[:, :WIDTH] / jnp.maximum(t1[:, WIDTH:WIDTH + 1], 1.0)
    agg2 = t2[:, :WIDTH] / jnp.maximum(t2[:, WIDTH:WIDTH + 1], 1.0)
    t = (agg1 + jnp.dot(h, r1_ref[...], preferred_element_type=f32) + z1_ref[...]
         + agg2 + jnp.dot(h, r2_ref[...], preferred_element_type=f32) + z2_ref[...])
    out_ref[...] = _wide(jnp.maximum(t, 0.0) + x0_ref[:, :WIDTH])


def _comb_call(h, x0, s1, s2, r1, z1, r2, z2):
    full = lambda a: pl.BlockSpec(a.shape, lambda i: tuple(0 for _ in a.shape))
    rowh = pl.BlockSpec((ROWT, HW), lambda i: (i, 0))
    args = (h, x0, s1[0], s1[1], s2[0], s2[1], r1, z1[None], r2, z2[None])
    return _pallas_call(
        _comb_body,
        grid=(N // ROWT,),
        in_specs=[rowh, rowh, rowh, rowh, rowh, rowh]
                 + [full(a) for a in args[6:]],
        out_specs=rowh,
        out_shape=jax.ShapeDtypeStruct((N, HW), jnp.float32),
    )(*args)


def _epi_body(h_ref, w2_ref, b2_ref, w3_ref, b3_ref, out_ref):
    f32 = jnp.float32
    t = jnp.maximum(jnp.dot(h_ref[:, :WIDTH], w2_ref[...],
                            preferred_element_type=f32) + b2_ref[...], 0.0)
    out_ref[...] = jnp.dot(t, w3_ref[...], preferred_element_type=f32) + b3_ref[...]


def _epi_call(h, w2, b2, w3, b3):
    full = lambda a: pl.BlockSpec(a.shape, lambda i: tuple(0 for _ in a.shape))
    args = (h, w2, b2[None], w3, b3[None])
    return _pallas_call(
        _epi_body,
        grid=(N // ROWT,),
        in_specs=[pl.BlockSpec((ROWT, HW), lambda i: (i, 0))]
                 + [full(a) for a in args[1:]],
        out_specs=pl.BlockSpec((ROWT, 1), lambda i: (i, 0)),
        out_shape=jax.ShapeDtypeStruct((N, 1), jnp.float32),
    )(*args)


def _prep_edges(edge_index):
    e0 = edge_index.shape[1]
    e_pad = -(-e0 // (NW * 128)) * (NW * 128)
    src = jnp.concatenate(
        [edge_index[0], jnp.zeros((e_pad - e0,), jnp.int32)])
    dst = jnp.concatenate(
        [edge_index[1], jnp.full((e_pad - e0,), N, jnp.int32)])
    return e_pad, src, dst


def kernel(x, edge_attr, edge_attr_boundary, params, edge_index,
           edge_index_boundary):
    p = params
    f32 = jnp.float32

    e_pad1, src1, dst1 = _prep_edges(edge_index)
    e_pad2, src2, dst2 = _prep_edges(edge_index_boundary)
    ea1 = jnp.concatenate(
        [edge_attr, jnp.zeros((e_pad1 - edge_attr.shape[0], KER_IN), f32)])
    ea2 = jnp.concatenate(
        [edge_attr_boundary,
         jnp.zeros((e_pad2 - edge_attr_boundary.shape[0], KER_IN), f32)])

    rm = (jnp.arange(32 * WIDTH)[None, :] // WIDTH
          == jnp.arange(HW)[:, None]).astype(f32)
    sm = (jnp.arange(128)[:, None] % WIDTH
          == jnp.arange(WIDTH)[None, :]).astype(f32)

    x0 = _x0_call(x, p["fc1_w"], p["fc1_b"])
    h = x0
    for _ in range(DEPTH):
        xg1 = _sc_gather(h, src1)
        xg2 = _sc_gather(h, src2)
        m1 = _msg_call(ea1, xg1, p["ker1"], rm, sm)
        m2 = _msg_call(ea2, xg2, p["ker2"], rm, sm)
        s1 = _sc_scatter_add(m1, dst1)
        s2 = _sc_scatter_add(m2, dst2)
        h = _comb_call(h, x0, s1, s2,
                       p["root1"], p["bias1"], p["root2"], p["bias2"])

    return _epi_call(h, p["fc2_w"], p["fc2_b"], p["fc3_w"], p["fc3_b"])
